# Initial kernel scaffold; baseline (speedup 1.0000x reference)
import jax
import jax.numpy as jnp
from jax.experimental import pallas as pl


def kernel(input_ids, tables):
    raise NotImplementedError("write your pallas kernel here")


# speedup vs baseline: 1.4131x; 1.4131x over previous
import functools

import jax
import jax.numpy as jnp
from jax import lax
from jax.experimental import pallas as pl
from jax.experimental.pallas import tpu as pltpu
from jax.experimental.pallas import tpu_sc as plsc

N_LAYERS = 6
VOCAB = 100000
HIDDEN = 64
B = 1024
L = 50
TOK = B * L
NW = 32
PER_W = TOK // NW


def _emb_body(ids_hbm, tab_hbm, out_hbm, idx_v, rows_v, sem):
    wid = lax.axis_index("s") * 2 + lax.axis_index("c")
    base = wid * PER_W
    pltpu.sync_copy(ids_hbm.at[pl.ds(base, PER_W)], idx_v)
    for layer in range(N_LAYERS):
        pltpu.async_copy(tab_hbm.at[layer].at[idx_v], rows_v, sem).wait()
        pltpu.sync_copy(rows_v, out_hbm.at[layer, pl.ds(base, PER_W)])
        pltpu.sync_copy(rows_v, out_hbm.at[2 * N_LAYERS - 1 - layer,
                                           pl.ds(base, PER_W)])


@functools.partial(
    pl.kernel,
    mesh=plsc.VectorSubcoreMesh(core_axis_name="c", subcore_axis_name="s"),
    compiler_params=pltpu.CompilerParams(use_tc_tiling_on_sc=False),
    out_type=jax.ShapeDtypeStruct((2 * N_LAYERS, TOK, HIDDEN), jnp.float32),
    scratch_types=[
        pltpu.VMEM((PER_W,), jnp.int32),
        pltpu.VMEM((PER_W, HIDDEN), jnp.float32),
        pltpu.SemaphoreType.DMA,
    ],
)
def _emb_kernel(ids_hbm, tab_hbm, out_hbm, idx_v, rows_v, sem):
    _emb_body(ids_hbm, tab_hbm, out_hbm, idx_v, rows_v, sem)


def kernel(input_ids, tables):
    ids_flat = input_ids.reshape(TOK)
    out = _emb_kernel(ids_flat, tables)
    return out.reshape(2 * N_LAYERS, B, L, HIDDEN)
